# baseline (speedup 1.0000x reference)
import functools

import jax
import jax.numpy as jnp
from jax import lax
from jax.experimental import pallas as pl
from jax.experimental.pallas import tpu as pltpu
from jax.experimental.pallas import tpu_sc as plsc

N = 10000
E = 160000
D = 256
HD = 128
LAYERS = 4
NC, NS = 2, 16
NW = NC * NS
C = 128
KPT = 80
EPAD = NS * KPT * C
NCHUNK = EPAD // C
ACC_N = 10240
RPT = ACC_N // NS
ZR = RPT // 4
NBUF = 4
MB = 1000
NBLK = N // MB

_sc_mesh = plsc.VectorSubcoreMesh(
    core_axis_name="c", subcore_axis_name="s", num_cores=NC, num_subcores=NS)


NP = 5
NQH = N // NP
NQ = 2048
RPQ = NQ // NS
ZRQ = RPQ // 4


@functools.partial(
    pl.kernel,
    out_type=jax.ShapeDtypeStruct((NC, NP, NQ, HD), jnp.float32),
    mesh=_sc_mesh,
    scratch_types=[
        pltpu.VMEM((KPT, C), jnp.int32),
        pltpu.VMEM((KPT, C), jnp.int32),
        pltpu.VMEM((KPT, C), jnp.int32),
        pltpu.VMEM((NBUF, C, HD), jnp.float32),
        pltpu.VMEM((ZRQ, HD), jnp.float32),
        pltpu.VMEM_SHARED((NQ, HD), jnp.float32),
        pltpu.SemaphoreType.DMA,
        pltpu.SemaphoreType.DMA,
        pltpu.SemaphoreType.DMA,
        pltpu.SemaphoreType.DMA,
        pltpu.SemaphoreType.DMA,
        pltpu.SemaphoreType.DMA,
        pltpu.SemaphoreType.DMA,
        pltpu.SemaphoreType.DMA,
    ],
)
def _seg_sum_sc(h2, src2, dst2, out, src_v, dst_v, dstq_v, rows_v, zero_v,
                acc_sh, gs0, gs1, gs2, gs3, ss0, ss1, ss2, ss3):
    gsems = (gs0, gs1, gs2, gs3)
    ssems = (ss0, ss1, ss2, ss3)
    c = lax.axis_index("c")
    s = lax.axis_index("s")

    pltpu.sync_copy(src2.at[pl.ds(s * KPT, KPT)], src_v)
    pltpu.sync_copy(dst2.at[pl.ds(s * KPT, KPT)], dst_v)

    nsub = C // 16
    cN = c * N

    def _adj(i, _):
        k = i // nsub
        off = (i % nsub) * 16
        src_v[k, pl.ds(off, 16)] = src_v[k, pl.ds(off, 16)] + cN
        return 0

    lax.fori_loop(0, KPT * nsub, _adj, 0)

    zf = jnp.zeros((16,), jnp.float32)

    def _zfill(i, _):
        zero_v[i // (HD // 16), pl.ds((i % (HD // 16)) * 16, 16)] = zf
        return 0

    lax.fori_loop(0, ZRQ * (HD // 16), _zfill, 0)

    def _main(g, _):
        for bq in range(NBUF):
            k = g * NBUF + bq
            buf = rows_v.at[bq]
            pltpu.make_async_copy(h2.at[src_v.at[k]], buf, gsems[bq]).wait()
            pltpu.async_copy(buf, acc_sh.at[dstq_v.at[k]], ssems[bq],
                             add=True)
            j = k + 2
            bj = (bq + 2) % NBUF
            bufj = rows_v.at[bj]

            @pl.when(j < KPT)
            def _():
                @pl.when(j >= NBUF)
                def _():
                    pltpu.make_async_copy(
                        bufj, acc_sh.at[dstq_v.at[j - NBUF]],
                        ssems[bj]).wait()

                pltpu.async_copy(h2.at[src_v.at[j]], bufj, gsems[bj])
        return 0

    for q in range(NP):
        base = q * NQH

        def _remap(i, _, base=base):
            k = i // nsub
            off = (i % nsub) * 16
            dq = dst_v[k, pl.ds(off, 16)] - base
            ok = (dq >= 0) & (dq < NQH)
            dstq_v[k, pl.ds(off, 16)] = jnp.where(ok, dq, NQH)
            return 0

        lax.fori_loop(0, KPT * nsub, _remap, 0)
        for j in range(4):
            pltpu.sync_copy(zero_v, acc_sh.at[pl.ds(s * RPQ + j * ZRQ, ZRQ)])
        plsc.subcore_barrier()
        for bq in range(2):
            pltpu.async_copy(h2.at[src_v.at[bq]], rows_v.at[bq], gsems[bq])
        lax.fori_loop(0, KPT // NBUF, _main, 0)
        for i in range(NBUF):
            k = KPT - NBUF + i
            pltpu.make_async_copy(rows_v.at[k % NBUF],
                                  acc_sh.at[dstq_v.at[k]],
                                  ssems[k % NBUF]).wait()
        plsc.subcore_barrier()
        pltpu.sync_copy(acc_sh.at[pl.ds(s * RPQ, RPQ)],
                        out.at[c, q, pl.ds(s * RPQ, RPQ)])


def _prep_body(x_ref, fm_ref, fs_ref, h2_ref):
    xb = x_ref[...]
    hn = (xb - fm_ref[...]) / fs_ref[...]
    h2_ref[0] = hn[:, :HD]
    h2_ref[1] = hn[:, HD:]


def _layer_body(h2_ref, seg_ref, dinv_ref, ws_ref, wn_ref, b_ref, g_ref,
                be_ref, out_ref):
    dinv = 1.0 / jnp.maximum(dinv_ref[0, 0], 1.0)
    f32 = jnp.float32
    z = (jnp.dot(h2_ref[0], ws_ref[0], preferred_element_type=f32)
         + jnp.dot(h2_ref[1], ws_ref[1], preferred_element_type=f32)
         + jnp.dot(seg_ref[0, 0] * dinv, wn_ref[0], preferred_element_type=f32)
         + jnp.dot(seg_ref[1, 0] * dinv, wn_ref[1], preferred_element_type=f32)
         + b_ref[...])
    mu = jnp.mean(z, axis=1, keepdims=True)
    var = jnp.mean((z - mu) ** 2, axis=1, keepdims=True)
    h = (z - mu) * lax.rsqrt(var + 1e-5) * g_ref[...] + be_ref[...]
    h = jnp.maximum(h, 0.0)
    out_ref[0] = h[:, :HD]
    out_ref[1] = h[:, HD:]


def _head_body(h2_ref, w1_ref, g1_ref, b1_ref, w2_ref, g2_ref, b2_ref,
               out_ref):
    f32 = jnp.float32
    z = (jnp.dot(h2_ref[0], w1_ref[0], preferred_element_type=f32)
         + jnp.dot(h2_ref[1], w1_ref[1], preferred_element_type=f32))
    mu = jnp.mean(z, axis=1, keepdims=True)
    var = jnp.mean((z - mu) ** 2, axis=1, keepdims=True)
    t = (z - mu) * lax.rsqrt(var + 1e-5) * g1_ref[...] + b1_ref[...]
    t = jnp.maximum(t, 0.0)
    z2 = jnp.dot(t, w2_ref[...], preferred_element_type=f32)
    mu2 = jnp.mean(z2, axis=1, keepdims=True)
    var2 = jnp.mean((z2 - mu2) ** 2, axis=1, keepdims=True)
    o = (z2 - mu2) * lax.rsqrt(var2 + 1e-5) * g2_ref[...] + b2_ref[...]
    out_ref[...] = jnp.maximum(o, 0.0)


def _full(shape):
    n = len(shape)
    return pl.BlockSpec(shape, lambda m: (0,) * n)


_prep = pl.pallas_call(
    _prep_body,
    grid=(NBLK,),
    in_specs=[
        pl.BlockSpec((MB, D), lambda m: (m, 0)),
        _full((1, D)),
        _full((1, D)),
    ],
    out_specs=pl.BlockSpec((NC, MB, HD), lambda m: (0, m, 0)),
    out_shape=jax.ShapeDtypeStruct((NC, N, HD), jnp.float32),
)

_layer = pl.pallas_call(
    _layer_body,
    grid=(NBLK,),
    in_specs=[
        pl.BlockSpec((NC, MB, HD), lambda m: (0, m, 0)),
        pl.BlockSpec((NC, 1, MB, HD), lambda m: (0, m // 2, m % 2, 0)),
        pl.BlockSpec((NC, 1, MB, HD), lambda m: (0, m // 2, m % 2, 0)),
        _full((NC, HD, D)),
        _full((NC, HD, D)),
        _full((1, D)),
        _full((1, D)),
        _full((1, D)),
    ],
    out_specs=pl.BlockSpec((NC, MB, HD), lambda m: (0, m, 0)),
    out_shape=jax.ShapeDtypeStruct((NC, N, HD), jnp.float32),
)

_head = pl.pallas_call(
    _head_body,
    grid=(NBLK,),
    in_specs=[
        pl.BlockSpec((NC, MB, HD), lambda m: (0, m, 0)),
        _full((NC, HD, D)),
        _full((1, D)),
        _full((1, D)),
        _full((D, HD)),
        _full((1, HD)),
        _full((1, HD)),
    ],
    out_specs=pl.BlockSpec((MB, HD), lambda m: (m, 0)),
    out_shape=jax.ShapeDtypeStruct((N, HD), jnp.float32),
)


def kernel(x, edge_index, selected, feat_mean, feat_std, W_self, W_neigh, b,
           ln_g, ln_b, W1, W2, ln2_g, ln2_b):
    del selected
    pad = EPAD - E
    src2 = jnp.pad(edge_index[0], (0, pad)).reshape(NCHUNK, C)
    dst2 = jnp.pad(edge_index[1], (0, pad),
                   constant_values=N).reshape(NCHUNK, C)

    h0 = _prep(x, feat_mean.reshape(1, D), feat_std.reshape(1, D))

    idx = jnp.array([0, 0, 1, 2, 3])
    first = jnp.array([1, 0, 0, 0, 0], jnp.int32)
    wts = (W_self[idx].reshape(LAYERS + 1, NC, HD, D),
           W_neigh[idx].reshape(LAYERS + 1, NC, HD, D),
           b[idx].reshape(LAYERS + 1, 1, D),
           ln_g[idx].reshape(LAYERS + 1, 1, D),
           ln_b[idx].reshape(LAYERS + 1, 1, D),
           first)

    def _step(carry, xs):
        tab, deg = carry
        ws, wn, bi, gi, bei, fl = xs
        seg = _seg_sum_sc(tab.reshape(NC * N, HD), src2, dst2)
        deg = jnp.where(fl > 0, seg, deg)
        hn = _layer(tab, seg, deg, ws, wn, bi, gi, bei)
        tab = jnp.where(fl > 0, h0, hn)
        return (tab, deg), None

    ones_tab = jnp.ones((NC, N, HD), jnp.float32)
    deg0 = jnp.ones((NC, NP, NQ, HD), jnp.float32)
    (tab, _), _ = lax.scan(_step, (ones_tab, deg0), wts)

    return _head(tab, W1.reshape(NC, HD, D), ln_g[LAYERS].reshape(1, D),
                 ln_b[LAYERS].reshape(1, D), W2, ln2_g.reshape(1, HD),
                 ln2_b.reshape(1, HD))


# speedup vs baseline: 1.0947x; 1.0947x over previous
import functools

import jax
import jax.numpy as jnp
from jax import lax
from jax.experimental import pallas as pl
from jax.experimental.pallas import tpu as pltpu
from jax.experimental.pallas import tpu_sc as plsc

N = 10000
E = 160000
D = 256
HD = 128
LAYERS = 4
NC, NS = 2, 16
NW = NC * NS
C = 128
KPT = 80
EPAD = NS * KPT * C
NCHUNK = EPAD // C
ACC_N = 10240
RPT = ACC_N // NS
ZR = RPT // 4
NBUF = 4
MB = 1000
NBLK = N // MB

_sc_mesh = plsc.VectorSubcoreMesh(
    core_axis_name="c", subcore_axis_name="s", num_cores=NC, num_subcores=NS)


NP = 4
NQH = 3000
NQ = 3072
RPQ = NQ // NS
ZRQ = RPQ // 4


@functools.partial(
    pl.kernel,
    out_type=jax.ShapeDtypeStruct((NC, NP, NQ, HD), jnp.float32),
    mesh=_sc_mesh,
    scratch_types=[
        pltpu.VMEM((KPT, C), jnp.int32),
        pltpu.VMEM((KPT, C), jnp.int32),
        pltpu.VMEM((KPT, C), jnp.int32),
        pltpu.VMEM((NBUF, C, HD), jnp.float32),
        pltpu.VMEM((ZRQ, HD), jnp.float32),
        pltpu.VMEM_SHARED((NQ, HD), jnp.float32),
        pltpu.SemaphoreType.DMA,
        pltpu.SemaphoreType.DMA,
        pltpu.SemaphoreType.DMA,
        pltpu.SemaphoreType.DMA,
        pltpu.SemaphoreType.DMA,
        pltpu.SemaphoreType.DMA,
        pltpu.SemaphoreType.DMA,
        pltpu.SemaphoreType.DMA,
    ],
)
def _seg_sum_sc(h2, src2, dst2, out, src_v, dst_v, dstq_v, rows_v, zero_v,
                acc_sh, gs0, gs1, gs2, gs3, ss0, ss1, ss2, ss3):
    gsems = (gs0, gs1, gs2, gs3)
    ssems = (ss0, ss1, ss2, ss3)
    c = lax.axis_index("c")
    s = lax.axis_index("s")

    pltpu.sync_copy(src2.at[pl.ds(s * KPT, KPT)], src_v)
    pltpu.sync_copy(dst2.at[pl.ds(s * KPT, KPT)], dst_v)

    nsub = C // 16
    cN = c * N

    def _adj(i, _):
        k = i // nsub
        off = (i % nsub) * 16
        src_v[k, pl.ds(off, 16)] = src_v[k, pl.ds(off, 16)] + cN
        return 0

    lax.fori_loop(0, KPT * nsub, _adj, 0)

    zf = jnp.zeros((16,), jnp.float32)

    def _zfill(i, _):
        zero_v[i // (HD // 16), pl.ds((i % (HD // 16)) * 16, 16)] = zf
        return 0

    lax.fori_loop(0, ZRQ * (HD // 16), _zfill, 0)

    def _main(g, _):
        for bq in range(NBUF):
            k = g * NBUF + bq
            buf = rows_v.at[bq]
            pltpu.make_async_copy(h2.at[src_v.at[k]], buf, gsems[bq]).wait()
            pltpu.async_copy(buf, acc_sh.at[dstq_v.at[k]], ssems[bq],
                             add=True)
            j = k + 2
            bj = (bq + 2) % NBUF
            bufj = rows_v.at[bj]

            @pl.when(j < KPT)
            def _():
                @pl.when(j >= NBUF)
                def _():
                    pltpu.make_async_copy(
                        bufj, acc_sh.at[dstq_v.at[j - NBUF]],
                        ssems[bj]).wait()

                pltpu.async_copy(h2.at[src_v.at[j]], bufj, gsems[bj])
        return 0

    for q in range(NP):
        base = q * NQH

        def _remap(i, _, base=base):
            k = i // nsub
            off = (i % nsub) * 16
            dq = dst_v[k, pl.ds(off, 16)] - base
            ok = (dq >= 0) & (dq < NQH)
            dstq_v[k, pl.ds(off, 16)] = jnp.where(ok, dq, NQH)
            return 0

        lax.fori_loop(0, KPT * nsub, _remap, 0)
        for j in range(4):
            pltpu.sync_copy(zero_v, acc_sh.at[pl.ds(s * RPQ + j * ZRQ, ZRQ)])
        plsc.subcore_barrier()
        for bq in range(2):
            pltpu.async_copy(h2.at[src_v.at[bq]], rows_v.at[bq], gsems[bq])
        lax.fori_loop(0, KPT // NBUF, _main, 0)
        for i in range(NBUF):
            k = KPT - NBUF + i
            pltpu.make_async_copy(rows_v.at[k % NBUF],
                                  acc_sh.at[dstq_v.at[k]],
                                  ssems[k % NBUF]).wait()
        plsc.subcore_barrier()
        pltpu.sync_copy(acc_sh.at[pl.ds(s * RPQ, RPQ)],
                        out.at[c, q, pl.ds(s * RPQ, RPQ)])


def _prep_body(x_ref, fm_ref, fs_ref, h2_ref):
    xb = x_ref[...]
    hn = (xb - fm_ref[...]) / fs_ref[...]
    h2_ref[0] = hn[:, :HD]
    h2_ref[1] = hn[:, HD:]


def _layer_body(h2_ref, seg_ref, dinv_ref, ws_ref, wn_ref, b_ref, g_ref,
                be_ref, out_ref):
    dinv = 1.0 / jnp.maximum(dinv_ref[0, 0], 1.0)
    f32 = jnp.float32
    z = (jnp.dot(h2_ref[0], ws_ref[0], preferred_element_type=f32)
         + jnp.dot(h2_ref[1], ws_ref[1], preferred_element_type=f32)
         + jnp.dot(seg_ref[0, 0] * dinv, wn_ref[0], preferred_element_type=f32)
         + jnp.dot(seg_ref[1, 0] * dinv, wn_ref[1], preferred_element_type=f32)
         + b_ref[...])
    mu = jnp.mean(z, axis=1, keepdims=True)
    var = jnp.mean((z - mu) ** 2, axis=1, keepdims=True)
    h = (z - mu) * lax.rsqrt(var + 1e-5) * g_ref[...] + be_ref[...]
    h = jnp.maximum(h, 0.0)
    out_ref[0] = h[:, :HD]
    out_ref[1] = h[:, HD:]


def _head_body(h2_ref, w1_ref, g1_ref, b1_ref, w2_ref, g2_ref, b2_ref,
               out_ref):
    f32 = jnp.float32
    z = (jnp.dot(h2_ref[0], w1_ref[0], preferred_element_type=f32)
         + jnp.dot(h2_ref[1], w1_ref[1], preferred_element_type=f32))
    mu = jnp.mean(z, axis=1, keepdims=True)
    var = jnp.mean((z - mu) ** 2, axis=1, keepdims=True)
    t = (z - mu) * lax.rsqrt(var + 1e-5) * g1_ref[...] + b1_ref[...]
    t = jnp.maximum(t, 0.0)
    z2 = jnp.dot(t, w2_ref[...], preferred_element_type=f32)
    mu2 = jnp.mean(z2, axis=1, keepdims=True)
    var2 = jnp.mean((z2 - mu2) ** 2, axis=1, keepdims=True)
    o = (z2 - mu2) * lax.rsqrt(var2 + 1e-5) * g2_ref[...] + b2_ref[...]
    out_ref[...] = jnp.maximum(o, 0.0)


def _full(shape):
    n = len(shape)
    return pl.BlockSpec(shape, lambda m: (0,) * n)


_prep = pl.pallas_call(
    _prep_body,
    grid=(NBLK,),
    in_specs=[
        pl.BlockSpec((MB, D), lambda m: (m, 0)),
        _full((1, D)),
        _full((1, D)),
    ],
    out_specs=pl.BlockSpec((NC, MB, HD), lambda m: (0, m, 0)),
    out_shape=jax.ShapeDtypeStruct((NC, N, HD), jnp.float32),
)

_layer = pl.pallas_call(
    _layer_body,
    grid=(NBLK,),
    in_specs=[
        pl.BlockSpec((NC, MB, HD), lambda m: (0, m, 0)),
        pl.BlockSpec((NC, 1, MB, HD), lambda m: (0, m // 3, m % 3, 0)),
        pl.BlockSpec((NC, 1, MB, HD), lambda m: (0, m // 3, m % 3, 0)),
        _full((NC, HD, D)),
        _full((NC, HD, D)),
        _full((1, D)),
        _full((1, D)),
        _full((1, D)),
    ],
    out_specs=pl.BlockSpec((NC, MB, HD), lambda m: (0, m, 0)),
    out_shape=jax.ShapeDtypeStruct((NC, N, HD), jnp.float32),
)

_head = pl.pallas_call(
    _head_body,
    grid=(NBLK,),
    in_specs=[
        pl.BlockSpec((NC, MB, HD), lambda m: (0, m, 0)),
        _full((NC, HD, D)),
        _full((1, D)),
        _full((1, D)),
        _full((D, HD)),
        _full((1, HD)),
        _full((1, HD)),
    ],
    out_specs=pl.BlockSpec((MB, HD), lambda m: (m, 0)),
    out_shape=jax.ShapeDtypeStruct((N, HD), jnp.float32),
)


def kernel(x, edge_index, selected, feat_mean, feat_std, W_self, W_neigh, b,
           ln_g, ln_b, W1, W2, ln2_g, ln2_b):
    del selected
    pad = EPAD - E
    src2 = jnp.pad(edge_index[0], (0, pad)).reshape(NCHUNK, C)
    dst2 = jnp.pad(edge_index[1], (0, pad),
                   constant_values=N).reshape(NCHUNK, C)

    h0 = _prep(x, feat_mean.reshape(1, D), feat_std.reshape(1, D))

    idx = jnp.array([0, 0, 1, 2, 3])
    first = jnp.array([1, 0, 0, 0, 0], jnp.int32)
    wts = (W_self[idx].reshape(LAYERS + 1, NC, HD, D),
           W_neigh[idx].reshape(LAYERS + 1, NC, HD, D),
           b[idx].reshape(LAYERS + 1, 1, D),
           ln_g[idx].reshape(LAYERS + 1, 1, D),
           ln_b[idx].reshape(LAYERS + 1, 1, D),
           first)

    def _step(carry, xs):
        tab, deg = carry
        ws, wn, bi, gi, bei, fl = xs
        seg = _seg_sum_sc(tab.reshape(NC * N, HD), src2, dst2)
        deg = jnp.where(fl > 0, seg, deg)
        hn = _layer(tab, seg, deg, ws, wn, bi, gi, bei)
        tab = jnp.where(fl > 0, h0, hn)
        return (tab, deg), None

    ones_tab = jnp.ones((NC, N, HD), jnp.float32)
    deg0 = jnp.ones((NC, NP, NQ, HD), jnp.float32)
    (tab, _), _ = lax.scan(_step, (ones_tab, deg0), wts)

    return _head(tab, W1.reshape(NC, HD, D), ln_g[LAYERS].reshape(1, D),
                 ln_b[LAYERS].reshape(1, D), W2, ln2_g.reshape(1, HD),
                 ln2_b.reshape(1, HD))


# speedup vs baseline: 1.0952x; 1.0004x over previous
import functools

import jax
import jax.numpy as jnp
from jax import lax
from jax.experimental import pallas as pl
from jax.experimental.pallas import tpu as pltpu
from jax.experimental.pallas import tpu_sc as plsc

N = 10000
E = 160000
D = 256
HD = 128
LAYERS = 4
NC, NS = 2, 16
NW = NC * NS
C = 128
KPT = 80
EPAD = NS * KPT * C
NCHUNK = EPAD // C
ACC_N = 10240
RPT = ACC_N // NS
ZR = RPT // 4
NBUF = 4
MB = 1000
NBLK = N // MB

_sc_mesh = plsc.VectorSubcoreMesh(
    core_axis_name="c", subcore_axis_name="s", num_cores=NC, num_subcores=NS)


NP = 4
NQH = 3000
NQ = 3072
RPQ = NQ // NS
ZRQ = RPQ // 4


@functools.partial(
    pl.kernel,
    out_type=jax.ShapeDtypeStruct((NC, NP, NQ, HD), jnp.float32),
    mesh=_sc_mesh,
    scratch_types=[
        pltpu.VMEM((KPT, C), jnp.int32),
        pltpu.VMEM((KPT, C), jnp.int32),
        pltpu.VMEM((KPT, C), jnp.int32),
        pltpu.VMEM((NBUF, C, HD), jnp.float32),
        pltpu.VMEM((ZRQ, HD), jnp.float32),
        pltpu.VMEM_SHARED((NQ, HD), jnp.float32),
        pltpu.SemaphoreType.DMA,
        pltpu.SemaphoreType.DMA,
        pltpu.SemaphoreType.DMA,
        pltpu.SemaphoreType.DMA,
        pltpu.SemaphoreType.DMA,
        pltpu.SemaphoreType.DMA,
        pltpu.SemaphoreType.DMA,
        pltpu.SemaphoreType.DMA,
    ],
)
def _seg_sum_sc(h2, src2, dst2, out, src_v, dst_v, dstq_v, rows_v, zero_v,
                acc_sh, gs0, gs1, gs2, gs3, ss0, ss1, ss2, ss3):
    gsems = (gs0, gs1, gs2, gs3)
    ssems = (ss0, ss1, ss2, ss3)
    c = lax.axis_index("c")
    s = lax.axis_index("s")

    pltpu.sync_copy(src2.at[pl.ds(s * KPT, KPT)], src_v)
    pltpu.sync_copy(dst2.at[pl.ds(s * KPT, KPT)], dst_v)

    nsub = C // 16
    cN = c * N

    def _adj(i, _):
        k = i // nsub
        off = (i % nsub) * 16
        src_v[k, pl.ds(off, 16)] = src_v[k, pl.ds(off, 16)] + cN
        return 0

    lax.fori_loop(0, KPT * nsub, _adj, 0)

    zf = jnp.zeros((16,), jnp.float32)

    def _zfill(i, _):
        zero_v[i // (HD // 16), pl.ds((i % (HD // 16)) * 16, 16)] = zf
        return 0

    lax.fori_loop(0, ZRQ * (HD // 16), _zfill, 0)

    def _main(g, _):
        for bq in range(NBUF):
            k = g * NBUF + bq
            buf = rows_v.at[bq]
            pltpu.make_async_copy(h2.at[src_v.at[k]], buf, gsems[bq]).wait()
            pltpu.async_copy(buf, acc_sh.at[dstq_v.at[k]], ssems[bq],
                             add=True)
            j = k + 3
            bj = (bq + 3) % NBUF
            bufj = rows_v.at[bj]

            @pl.when(j < KPT)
            def _():
                @pl.when(j >= NBUF)
                def _():
                    pltpu.make_async_copy(
                        bufj, acc_sh.at[dstq_v.at[j - NBUF]],
                        ssems[bj]).wait()

                pltpu.async_copy(h2.at[src_v.at[j]], bufj, gsems[bj])
        return 0

    for q in range(NP):
        base = q * NQH

        def _remap(i, _, base=base):
            k = i // nsub
            off = (i % nsub) * 16
            dq = dst_v[k, pl.ds(off, 16)] - base
            ok = (dq >= 0) & (dq < NQH)
            dstq_v[k, pl.ds(off, 16)] = jnp.where(ok, dq, NQH)
            return 0

        lax.fori_loop(0, KPT * nsub, _remap, 0)
        for j in range(4):
            pltpu.sync_copy(zero_v, acc_sh.at[pl.ds(s * RPQ + j * ZRQ, ZRQ)])
        plsc.subcore_barrier()
        for bq in range(3):
            pltpu.async_copy(h2.at[src_v.at[bq]], rows_v.at[bq], gsems[bq])
        lax.fori_loop(0, KPT // NBUF, _main, 0)
        for i in range(NBUF):
            k = KPT - NBUF + i
            pltpu.make_async_copy(rows_v.at[k % NBUF],
                                  acc_sh.at[dstq_v.at[k]],
                                  ssems[k % NBUF]).wait()
        plsc.subcore_barrier()
        pltpu.sync_copy(acc_sh.at[pl.ds(s * RPQ, RPQ)],
                        out.at[c, q, pl.ds(s * RPQ, RPQ)])


def _prep_body(x_ref, fm_ref, fs_ref, h2_ref):
    xb = x_ref[...]
    hn = (xb - fm_ref[...]) / fs_ref[...]
    h2_ref[0] = hn[:, :HD]
    h2_ref[1] = hn[:, HD:]


def _layer_body(h2_ref, seg_ref, dinv_ref, ws_ref, wn_ref, b_ref, g_ref,
                be_ref, out_ref):
    dinv = 1.0 / jnp.maximum(dinv_ref[0, 0], 1.0)
    f32 = jnp.float32
    z = (jnp.dot(h2_ref[0], ws_ref[0], preferred_element_type=f32)
         + jnp.dot(h2_ref[1], ws_ref[1], preferred_element_type=f32)
         + jnp.dot(seg_ref[0, 0] * dinv, wn_ref[0], preferred_element_type=f32)
         + jnp.dot(seg_ref[1, 0] * dinv, wn_ref[1], preferred_element_type=f32)
         + b_ref[...])
    mu = jnp.mean(z, axis=1, keepdims=True)
    var = jnp.mean((z - mu) ** 2, axis=1, keepdims=True)
    h = (z - mu) * lax.rsqrt(var + 1e-5) * g_ref[...] + be_ref[...]
    h = jnp.maximum(h, 0.0)
    out_ref[0] = h[:, :HD]
    out_ref[1] = h[:, HD:]


def _head_body(h2_ref, w1_ref, g1_ref, b1_ref, w2_ref, g2_ref, b2_ref,
               out_ref):
    f32 = jnp.float32
    z = (jnp.dot(h2_ref[0], w1_ref[0], preferred_element_type=f32)
         + jnp.dot(h2_ref[1], w1_ref[1], preferred_element_type=f32))
    mu = jnp.mean(z, axis=1, keepdims=True)
    var = jnp.mean((z - mu) ** 2, axis=1, keepdims=True)
    t = (z - mu) * lax.rsqrt(var + 1e-5) * g1_ref[...] + b1_ref[...]
    t = jnp.maximum(t, 0.0)
    z2 = jnp.dot(t, w2_ref[...], preferred_element_type=f32)
    mu2 = jnp.mean(z2, axis=1, keepdims=True)
    var2 = jnp.mean((z2 - mu2) ** 2, axis=1, keepdims=True)
    o = (z2 - mu2) * lax.rsqrt(var2 + 1e-5) * g2_ref[...] + b2_ref[...]
    out_ref[...] = jnp.maximum(o, 0.0)


def _full(shape):
    n = len(shape)
    return pl.BlockSpec(shape, lambda m: (0,) * n)


_prep = pl.pallas_call(
    _prep_body,
    grid=(NBLK,),
    in_specs=[
        pl.BlockSpec((MB, D), lambda m: (m, 0)),
        _full((1, D)),
        _full((1, D)),
    ],
    out_specs=pl.BlockSpec((NC, MB, HD), lambda m: (0, m, 0)),
    out_shape=jax.ShapeDtypeStruct((NC, N, HD), jnp.float32),
)

_layer = pl.pallas_call(
    _layer_body,
    grid=(NBLK,),
    in_specs=[
        pl.BlockSpec((NC, MB, HD), lambda m: (0, m, 0)),
        pl.BlockSpec((NC, 1, MB, HD), lambda m: (0, m // 3, m % 3, 0)),
        pl.BlockSpec((NC, 1, MB, HD), lambda m: (0, m // 3, m % 3, 0)),
        _full((NC, HD, D)),
        _full((NC, HD, D)),
        _full((1, D)),
        _full((1, D)),
        _full((1, D)),
    ],
    out_specs=pl.BlockSpec((NC, MB, HD), lambda m: (0, m, 0)),
    out_shape=jax.ShapeDtypeStruct((NC, N, HD), jnp.float32),
)

_head = pl.pallas_call(
    _head_body,
    grid=(NBLK,),
    in_specs=[
        pl.BlockSpec((NC, MB, HD), lambda m: (0, m, 0)),
        _full((NC, HD, D)),
        _full((1, D)),
        _full((1, D)),
        _full((D, HD)),
        _full((1, HD)),
        _full((1, HD)),
    ],
    out_specs=pl.BlockSpec((MB, HD), lambda m: (m, 0)),
    out_shape=jax.ShapeDtypeStruct((N, HD), jnp.float32),
)


def kernel(x, edge_index, selected, feat_mean, feat_std, W_self, W_neigh, b,
           ln_g, ln_b, W1, W2, ln2_g, ln2_b):
    del selected
    pad = EPAD - E
    src2 = jnp.pad(edge_index[0], (0, pad)).reshape(NCHUNK, C)
    dst2 = jnp.pad(edge_index[1], (0, pad),
                   constant_values=N).reshape(NCHUNK, C)

    h0 = _prep(x, feat_mean.reshape(1, D), feat_std.reshape(1, D))

    idx = jnp.array([0, 0, 1, 2, 3])
    first = jnp.array([1, 0, 0, 0, 0], jnp.int32)
    wts = (W_self[idx].reshape(LAYERS + 1, NC, HD, D),
           W_neigh[idx].reshape(LAYERS + 1, NC, HD, D),
           b[idx].reshape(LAYERS + 1, 1, D),
           ln_g[idx].reshape(LAYERS + 1, 1, D),
           ln_b[idx].reshape(LAYERS + 1, 1, D),
           first)

    def _step(carry, xs):
        tab, deg = carry
        ws, wn, bi, gi, bei, fl = xs
        seg = _seg_sum_sc(tab.reshape(NC * N, HD), src2, dst2)
        deg = jnp.where(fl > 0, seg, deg)
        hn = _layer(tab, seg, deg, ws, wn, bi, gi, bei)
        tab = jnp.where(fl > 0, h0, hn)
        return (tab, deg), None

    ones_tab = jnp.ones((NC, N, HD), jnp.float32)
    deg0 = jnp.ones((NC, NP, NQ, HD), jnp.float32)
    (tab, _), _ = lax.scan(_step, (ones_tab, deg0), wts)

    return _head(tab, W1.reshape(NC, HD, D), ln_g[LAYERS].reshape(1, D),
                 ln_b[LAYERS].reshape(1, D), W2, ln2_g.reshape(1, HD),
                 ln2_b.reshape(1, HD))
